# baseline (speedup 1.0000x reference)
import functools

import jax
import jax.numpy as jnp
from jax import lax
from jax.experimental import pallas as pl
from jax.experimental.pallas import tpu as pltpu
from jax.experimental.pallas import tpu_sc as plsc

N = 10000
E = 320000
D = 128

NT = 16
CHUNK = 128
EPT = E // NT
NCH = 160
EPT_PAD = NCH * CHUNK
N_PAD = 10240
NPASS = 5
HALF = N_PAD // NPASS
HTRASH = 16
ROWS_PP = HALF // NT
ZR = 64
PMAX = NCH + 2


def _sc_seg_sum(x2n, src_pad, dst_pad, zeros_d, ones_d, with_cnt):
    mesh = plsc.VectorSubcoreMesh(core_axis_name="c", subcore_axis_name="s")

    out_type = [jax.ShapeDtypeStruct((2 * N_PAD, D), jnp.float32)]
    if with_cnt:
        out_type.append(jax.ShapeDtypeStruct((2 * N_PAD, D), jnp.float32))

    scratch = [
        pltpu.VMEM((16, CHUNK), jnp.int32),
        pltpu.VMEM((16, CHUNK), jnp.int32),
        pltpu.VMEM((PMAX, CHUNK), jnp.int32),
        pltpu.VMEM((PMAX, CHUNK), jnp.int32),
        pltpu.VMEM((CHUNK, D), jnp.float32),
        pltpu.VMEM((CHUNK, D), jnp.float32),
        pltpu.VMEM((ZR, D), jnp.float32),
        pltpu.VMEM((ZR, D), jnp.float32),
        pltpu.VMEM_SHARED((HALF + HTRASH, D), jnp.float32),
    ] + [pltpu.SemaphoreType.DMA] * 2

    @functools.partial(
        pl.kernel, out_type=tuple(out_type), mesh=mesh,
        scratch_types=scratch, name="sc_seg_sum",
        compiler_params=pltpu.CompilerParams(use_tc_tiling_on_sc=False,
                                            needs_layout_passes=False),
    )
    def k(x_hbm, src_hbm, dst_hbm, zd_hbm, o_hbm, *rest):
        if with_cnt:
            out_hbm, cnt_hbm = rest[0], rest[1]
            rest = rest[2:]
        else:
            out_hbm = rest[0]
            rest = rest[1:]
        (sblk, dblk, psrc, pdst, r0, r1, zb, stg, acc,
         sem0, sem1) = rest
        bufs = (r0, r1)
        gsem = (sem0, sem1)

        g = lax.axis_index("c")
        t = lax.axis_index("s")
        base = t * ROWS_PP

        pltpu.sync_copy(zd_hbm, zb)

        lanes = lax.iota(jnp.int32, 16)

        def zero_own_rows():
            for kk in range(ROWS_PP // ZR):
                pltpu.sync_copy(zb, acc.at[pl.ds(base + kk * ZR, ZR)])

        def dump_own_rows(dst_hbm_arr, lo):
            for kk in range(ROWS_PP // ZR):
                pltpu.sync_copy(acc.at[pl.ds(base + kk * ZR, ZR)], stg)
                pltpu.sync_copy(
                    stg,
                    dst_hbm_arr.at[pl.ds(g * N_PAD + lo + base + kk * ZR,
                                         ZR)])

        for p in range(NPASS):
            lo = p * HALF

            def cblock(blk, off):
                pltpu.sync_copy(src_hbm.at[g, t, pl.ds(blk * 16, 16)], sblk)
                pltpu.sync_copy(dst_hbm.at[g, t, pl.ds(blk * 16, 16)], dblk)

                def crow(r, off):
                    for c in range(CHUNK // 16):
                        s = sblk[r, pl.ds(c * 16, 16)]
                        d = dblk[r, pl.ds(c * 16, 16)]
                        dr = d - lo
                        m = (dr >= 0) & (dr < HALF)
                        inc = plsc.cumsum(jnp.where(m, 1, 0))
                        pos = off + inc - 1
                        row = lax.shift_right_logical(pos, 7)
                        col = lax.bitwise_and(pos, 127)
                        plsc.store_scatter(psrc, [row, col], s, mask=m)
                        plsc.store_scatter(pdst, [row, col], dr, mask=m)
                        off = off + inc[15]
                    return off

                return lax.fori_loop(0, 16, crow, off)

            nedge = lax.fori_loop(0, NCH // 16, cblock, jnp.int32(0))

            def tfill(i, off):
                pos = off + lanes
                row = lax.shift_right_logical(pos, 7)
                col = lax.bitwise_and(pos, 127)
                full = jnp.full((16,), 1, jnp.int32) > 0
                plsc.store_scatter(psrc, [row, col],
                                   jnp.full((16,), 1, jnp.int32) * g * N_PAD,
                                   mask=full)
                plsc.store_scatter(pdst, [row, col],
                                   jnp.full((16,), HALF, jnp.int32),
                                   mask=full)
                return off + 16

            lax.fori_loop(0, CHUNK // 16, tfill, nedge)
            nchunk = lax.shift_right_logical(nedge + (CHUNK - 1), 7)

            zero_own_rows()
            plsc.subcore_barrier()

            @pl.when(nchunk > 0)
            def _():
                pltpu.async_copy(x_hbm.at[psrc.at[0]], bufs[0], gsem[0])

            def gbody(i, carry):
                j0 = 2 * i

                @pl.when(j0 < nchunk)
                def _():
                    pltpu.make_async_copy(
                        x_hbm.at[pl.ds(0, CHUNK)], bufs[0], gsem[0]).wait()

                    @pl.when(j0 + 1 < nchunk)
                    def _():
                        pltpu.async_copy(
                            x_hbm.at[psrc.at[j0 + 1]], bufs[1], gsem[1])
                    pltpu.sync_copy(bufs[0], acc.at[pdst.at[j0]], add=True)

                @pl.when(j0 + 1 < nchunk)
                def _():
                    @pl.when(j0 + 2 < nchunk)
                    def _():
                        pltpu.async_copy(
                            x_hbm.at[psrc.at[j0 + 2]], bufs[0], gsem[0])
                    pltpu.make_async_copy(
                        x_hbm.at[pl.ds(0, CHUNK)], bufs[1], gsem[1]).wait()
                    pltpu.sync_copy(
                        bufs[1], acc.at[pdst.at[j0 + 1]], add=True)
                return carry

            lax.fori_loop(0, (NCH + 2) // 2, gbody, 0)
            plsc.subcore_barrier()
            dump_own_rows(out_hbm, lo)

            if with_cnt:
                pltpu.sync_copy(o_hbm, bufs[0])
                zero_own_rows()
                plsc.subcore_barrier()

                def cntbody(j, carry):
                    @pl.when(j < nchunk)
                    def _():
                        pltpu.sync_copy(bufs[0], acc.at[pdst.at[j]], add=True)
                    return carry

                lax.fori_loop(0, NCH + 1, cntbody, 0)
                plsc.subcore_barrier()
                dump_own_rows(cnt_hbm, lo)
            plsc.subcore_barrier()

    return k(x2n, src_pad, dst_pad, zeros_d, ones_d)


def _tc_layer(s, cnt, x, wlt, b, wrt):
    B = 1024
    NB = N_PAD // B

    def body(s_ref, c_ref, x_ref, wl_ref, b_ref, wr_ref, o_ref):
        rcp = 1.0 / jnp.maximum(c_ref[:, 0:1], 1.0)
        agg = jnp.dot(s_ref[...], wl_ref[0], preferred_element_type=jnp.float32)
        res = jnp.dot(x_ref[...], wr_ref[0], preferred_element_type=jnp.float32)
        o_ref[...] = jnp.tanh(agg * rcp + b_ref[0] + res)

    return pl.pallas_call(
        body,
        grid=(2, NB),
        in_specs=[
            pl.BlockSpec((B, D), lambda g, i: (g * NB + i, 0)),
            pl.BlockSpec((B, D), lambda g, i: (g * NB + i, 0)),
            pl.BlockSpec((B, D), lambda g, i: (g * NB + i, 0)),
            pl.BlockSpec((1, D, D), lambda g, i: (g, 0, 0)),
            pl.BlockSpec((1, 1, D), lambda g, i: (g, 0, 0)),
            pl.BlockSpec((1, D, D), lambda g, i: (g, 0, 0)),
        ],
        out_specs=pl.BlockSpec((B, D), lambda g, i: (g * NB + i, 0)),
        out_shape=jax.ShapeDtypeStruct((2 * N_PAD, D), jnp.float32),
    )(s, cnt, x, wlt, b, wrt)


def _pad_edges(ei, g):
    src = ei[0] + g * N_PAD
    dst = ei[1]
    src = src.reshape(NT, EPT)
    dst = dst.reshape(NT, EPT)
    pad = EPT_PAD - EPT
    src = jnp.pad(src, ((0, 0), (0, pad)), constant_values=g * N_PAD)
    dst = jnp.pad(dst, ((0, 0), (0, pad)), constant_values=N)
    return src.reshape(NT, NCH, CHUNK), dst.reshape(NT, NCH, CHUNK)


def kernel(x0, x1, edge_index0, edge_index1,
           g0_W1l, g0_b1, g0_W1r, g0_W2l, g0_b2, g0_W2r,
           g1_W1l, g1_b1, g1_W1r, g1_W2l, g1_b2, g1_W2r):
    pad = ((0, N_PAD - N), (0, 0))
    xs = jnp.concatenate([jnp.pad(x0, pad), jnp.pad(x1, pad)], axis=0)
    sg0, dg0 = _pad_edges(edge_index0, 0)
    sg1, dg1 = _pad_edges(edge_index1, 1)
    src_pad = jnp.stack([sg0, sg1])
    dst_pad = jnp.stack([dg0, dg1])

    zeros_d = jnp.zeros((ZR, D), jnp.float32)
    ones_d = jnp.ones((CHUNK, D), jnp.float32)

    w1lt = jnp.stack([g0_W1l.T, g1_W1l.T])
    w1rt = jnp.stack([g0_W1r.T, g1_W1r.T])
    b1 = jnp.stack([g0_b1, g1_b1])[:, None, :]
    w2lt = jnp.stack([g0_W2l.T, g1_W2l.T])
    w2rt = jnp.stack([g0_W2r.T, g1_W2r.T])
    b2 = jnp.stack([g0_b2, g1_b2])[:, None, :]

    s_1, cnt = _sc_seg_sum(xs, src_pad, dst_pad, zeros_d, ones_d,
                           with_cnt=True)
    h1 = _tc_layer(s_1, cnt, xs, w1lt, b1, w1rt)
    (s_2,) = _sc_seg_sum(h1, src_pad, dst_pad, zeros_d, ones_d,
                         with_cnt=False)
    out = _tc_layer(s_2, cnt, h1, w2lt, b2, w2rt)
    return out.reshape(2, N_PAD, D)[:, :N].reshape(2 * N, D)


# speedup vs baseline: 1.6330x; 1.6330x over previous
import functools

import jax
import jax.numpy as jnp
from jax import lax
from jax.experimental import pallas as pl
from jax.experimental.pallas import tpu as pltpu
from jax.experimental.pallas import tpu_sc as plsc

N = 10000
E = 320000
D = 128

NT = 16
CHUNK = 128
EPT = E // NT
NCH = 160
EPT_PAD = NCH * CHUNK
N_PAD = 10240
ROWS_PT = N_PAD // NT
ZR = 128


def _sc_seg_sum(x_half, src2_pad, dst_pad, zeros_d, zeros_16, ones_16,
                with_cnt):
    mesh = plsc.VectorSubcoreMesh(core_axis_name="c", subcore_axis_name="s")

    out_type = [jax.ShapeDtypeStruct((2 * N_PAD, 64), jnp.float32),
                jax.ShapeDtypeStruct((2 * N_PAD, 64), jnp.float32)]
    if with_cnt:
        out_type.append(jax.ShapeDtypeStruct((2 * N_PAD, 16), jnp.float32))

    scratch = [
        pltpu.VMEM((NCH, CHUNK), jnp.int32),
        pltpu.VMEM((NCH, CHUNK), jnp.int32),
        pltpu.VMEM((CHUNK, 64), jnp.float32),
        pltpu.VMEM((CHUNK, 64), jnp.float32),
        pltpu.VMEM((CHUNK, 16), jnp.float32),
        pltpu.VMEM((ZR, 64), jnp.float32),
        pltpu.VMEM((ZR, 16), jnp.float32),
        pltpu.VMEM((ZR, 64), jnp.float32),
        pltpu.VMEM((ZR, 16), jnp.float32),
        pltpu.VMEM_SHARED((N_PAD, 64), jnp.float32),
        pltpu.VMEM_SHARED((N_PAD, 16), jnp.float32),
    ] + [pltpu.SemaphoreType.DMA] * 4

    @functools.partial(
        pl.kernel, out_type=tuple(out_type), mesh=mesh,
        scratch_types=scratch, name="sc_seg_sum",
        compiler_params=pltpu.CompilerParams(use_tc_tiling_on_sc=False),
    )
    def k(x_hbm, src_hbm, dst_hbm, zd_hbm, z16_hbm, o16_hbm, *rest):
        if with_cnt:
            s_hbm = (rest[0], rest[1])
            cnt_hbm = rest[2]
            rest = rest[3:]
        else:
            s_hbm = (rest[0], rest[1])
            rest = rest[2:]
        (srcv, dstv, r0, r1, onesv, zb, zb16, stg, stg16,
         acc, accc, *sems) = rest
        bufs = [r0, r1]
        gsem = sems[:2]
        ssem = sems[2:]

        g = lax.axis_index("c")
        t = lax.axis_index("s")
        base = t * ROWS_PT

        pltpu.sync_copy(dst_hbm.at[g, t], dstv)
        pltpu.sync_copy(zd_hbm, zb)
        if with_cnt:
            pltpu.sync_copy(z16_hbm, zb16)
            pltpu.sync_copy(o16_hbm, onesv)

        for h in range(2):
            cnt_pass = with_cnt and h == 0
            pltpu.sync_copy(src_hbm.at[h, g, t], srcv)

            for kk in range(ROWS_PT // ZR):
                pltpu.sync_copy(zb, acc.at[pl.ds(base + kk * ZR, ZR)])
                if cnt_pass:
                    pltpu.sync_copy(zb16, accc.at[pl.ds(base + kk * ZR, ZR)])
            plsc.subcore_barrier()

            pltpu.async_copy(x_hbm.at[srcv.at[0]], bufs[0], gsem[0])

            def body(i, carry):
                j0 = 2 * i
                pltpu.make_async_copy(
                    x_hbm.at[pl.ds(0, CHUNK)], bufs[0], gsem[0]).wait()
                pltpu.async_copy(x_hbm.at[srcv.at[j0 + 1]], bufs[1], gsem[1])
                pltpu.sync_copy(bufs[0], acc.at[dstv.at[j0]], add=True)
                if cnt_pass:
                    pltpu.sync_copy(onesv, accc.at[dstv.at[j0]], add=True)

                @pl.when(j0 + 2 < NCH)
                def _():
                    pltpu.async_copy(
                        x_hbm.at[srcv.at[j0 + 2]], bufs[0], gsem[0])
                pltpu.make_async_copy(
                    x_hbm.at[pl.ds(0, CHUNK)], bufs[1], gsem[1]).wait()
                pltpu.sync_copy(bufs[1], acc.at[dstv.at[j0 + 1]], add=True)
                if cnt_pass:
                    pltpu.sync_copy(
                        onesv, accc.at[dstv.at[j0 + 1]], add=True)
                return carry

            lax.fori_loop(0, NCH // 2, body, 0)
            plsc.subcore_barrier()

            for kk in range(ROWS_PT // ZR):
                pltpu.sync_copy(acc.at[pl.ds(base + kk * ZR, ZR)], stg)
                pltpu.sync_copy(
                    stg, s_hbm[h].at[pl.ds(g * N_PAD + base + kk * ZR, ZR)])
                if cnt_pass:
                    pltpu.sync_copy(accc.at[pl.ds(base + kk * ZR, ZR)], stg16)
                    pltpu.sync_copy(
                        stg16,
                        cnt_hbm.at[pl.ds(g * N_PAD + base + kk * ZR, ZR)])

    return k(x_half, src2_pad, dst_pad, zeros_d, zeros_16, ones_16)


def _tc_layer(s0, s1, cnt, x, wlt, b, wrt):
    B = 1024
    NB = N_PAD // B

    def body(s0_ref, s1_ref, c_ref, x_ref, wl_ref, b_ref, wr_ref, o_ref):
        rcp = 1.0 / jnp.maximum(c_ref[:, 0:1], 1.0)
        agg = jnp.dot(s0_ref[...], wl_ref[0, :64],
                      preferred_element_type=jnp.float32)
        agg += jnp.dot(s1_ref[...], wl_ref[0, 64:],
                       preferred_element_type=jnp.float32)
        res = jnp.dot(x_ref[...], wr_ref[0], preferred_element_type=jnp.float32)
        o_ref[...] = jnp.tanh(agg * rcp + b_ref[0] + res)

    return pl.pallas_call(
        body,
        grid=(2, NB),
        in_specs=[
            pl.BlockSpec((B, 64), lambda g, i: (g * NB + i, 0)),
            pl.BlockSpec((B, 64), lambda g, i: (g * NB + i, 0)),
            pl.BlockSpec((B, 16), lambda g, i: (g * NB + i, 0)),
            pl.BlockSpec((B, D), lambda g, i: (g * NB + i, 0)),
            pl.BlockSpec((1, D, D), lambda g, i: (g, 0, 0)),
            pl.BlockSpec((1, 1, D), lambda g, i: (g, 0, 0)),
            pl.BlockSpec((1, D, D), lambda g, i: (g, 0, 0)),
        ],
        out_specs=pl.BlockSpec((B, D), lambda g, i: (g * NB + i, 0)),
        out_shape=jax.ShapeDtypeStruct((2 * N_PAD, D), jnp.float32),
    )(s0, s1, cnt, x, wlt, b, wrt)


def _pad_edges(ei, g):
    src2 = 2 * (ei[0] + g * N_PAD)
    dst = ei[1]
    src2 = src2.reshape(NT, EPT)
    dst = dst.reshape(NT, EPT)
    pad = EPT_PAD - EPT
    src2 = jnp.pad(src2, ((0, 0), (0, pad)), constant_values=2 * g * N_PAD)
    dst = jnp.pad(dst, ((0, 0), (0, pad)), constant_values=N)
    src2 = src2.reshape(NT, NCH, CHUNK)
    return jnp.stack([src2, src2 + 1]), dst.reshape(NT, NCH, CHUNK)


def kernel(x0, x1, edge_index0, edge_index1,
           g0_W1l, g0_b1, g0_W1r, g0_W2l, g0_b2, g0_W2r,
           g1_W1l, g1_b1, g1_W1r, g1_W2l, g1_b2, g1_W2r):
    pad = ((0, N_PAD - N), (0, 0))
    xs = jnp.concatenate([jnp.pad(x0, pad), jnp.pad(x1, pad)], axis=0)
    sg0, dg0 = _pad_edges(edge_index0, 0)
    sg1, dg1 = _pad_edges(edge_index1, 1)
    src2_pad = jnp.stack([sg0, sg1], axis=1)
    dst_pad = jnp.stack([dg0, dg1])

    zeros_d = jnp.zeros((ZR, 64), jnp.float32)
    zeros_16 = jnp.zeros((ZR, 16), jnp.float32)
    ones_16 = jnp.ones((CHUNK, 16), jnp.float32)

    w1lt = jnp.stack([g0_W1l.T, g1_W1l.T])
    w1rt = jnp.stack([g0_W1r.T, g1_W1r.T])
    b1 = jnp.stack([g0_b1, g1_b1])[:, None, :]
    w2lt = jnp.stack([g0_W2l.T, g1_W2l.T])
    w2rt = jnp.stack([g0_W2r.T, g1_W2r.T])
    b2 = jnp.stack([g0_b2, g1_b2])[:, None, :]

    xs_half = xs.reshape(4 * N_PAD, 64)
    sa, sb, cnt = _sc_seg_sum(xs_half, src2_pad, dst_pad, zeros_d, zeros_16,
                              ones_16, with_cnt=True)
    h1 = _tc_layer(sa, sb, cnt, xs, w1lt, b1, w1rt)
    sa2, sb2 = _sc_seg_sum(h1.reshape(4 * N_PAD, 64), src2_pad, dst_pad,
                           zeros_d, zeros_16, ones_16, with_cnt=False)
    out = _tc_layer(sa2, sb2, cnt, h1, w2lt, b2, w2rt)
    return out.reshape(2, N_PAD, D)[:, :N].reshape(2 * N, D)


# speedup vs baseline: 2.1944x; 1.3437x over previous
import functools

import jax
import jax.numpy as jnp
from jax import lax
from jax.experimental import pallas as pl
from jax.experimental.pallas import tpu as pltpu
from jax.experimental.pallas import tpu_sc as plsc

N = 10000
E = 320000
D = 128

NT = 16
CHUNK = 128
EPT = E // NT
NCH = 158
EPT_PAD = NCH * CHUNK
N_PAD = 10240
ROWS_PT = N_PAD // NT
ZR = 128


def _sc_seg_sum(x_half, src2_pad, dst_pad, zeros_d, zeros_16, ones_16,
                with_cnt):
    mesh = plsc.VectorSubcoreMesh(core_axis_name="c", subcore_axis_name="s")

    out_type = [jax.ShapeDtypeStruct((2 * N_PAD, 64), jnp.float32),
                jax.ShapeDtypeStruct((2 * N_PAD, 64), jnp.float32)]
    if with_cnt:
        out_type.append(jax.ShapeDtypeStruct((2 * N_PAD, 16), jnp.float32))

    scratch = [
        pltpu.VMEM((NCH, CHUNK), jnp.int32),
        pltpu.VMEM((NCH, CHUNK), jnp.int32),
        pltpu.VMEM((CHUNK, 64), jnp.float32),
        pltpu.VMEM((CHUNK, 64), jnp.float32),
        pltpu.VMEM((CHUNK, 16), jnp.float32),
        pltpu.VMEM((ZR, 64), jnp.float32),
        pltpu.VMEM((ZR, 16), jnp.float32),
        pltpu.VMEM((ZR, 64), jnp.float32),
        pltpu.VMEM((ZR, 16), jnp.float32),
        pltpu.VMEM_SHARED((N_PAD, 64), jnp.float32),
        pltpu.VMEM_SHARED((N_PAD, 16), jnp.float32),
    ] + [pltpu.SemaphoreType.DMA] * 2

    @functools.partial(
        pl.kernel, out_type=tuple(out_type), mesh=mesh,
        scratch_types=scratch, name="sc_seg_sum",
        compiler_params=pltpu.CompilerParams(use_tc_tiling_on_sc=False),
    )
    def k(x_hbm, src_hbm, dst_hbm, zd_hbm, z16_hbm, o16_hbm, *rest):
        if with_cnt:
            s_hbm = (rest[0], rest[1])
            cnt_hbm = rest[2]
            rest = rest[3:]
        else:
            s_hbm = (rest[0], rest[1])
            rest = rest[2:]
        (srcv, dstv, r0, r1, onesv, zb, zb16, stg, stg16,
         acc, accc, *sems) = rest
        bufs = [r0, r1]
        gsem = sems

        g = lax.axis_index("c")
        t = lax.axis_index("s")
        base = t * ROWS_PT

        pltpu.sync_copy(dst_hbm.at[g, t], dstv)
        pltpu.sync_copy(zd_hbm, zb)
        if with_cnt:
            pltpu.sync_copy(z16_hbm, zb16)
            pltpu.sync_copy(o16_hbm, onesv)

        for h in range(2):
            cnt_pass = with_cnt and h == 0
            pltpu.sync_copy(src_hbm.at[h, g, t], srcv)

            for kk in range(ROWS_PT // ZR):
                pltpu.sync_copy(zb, acc.at[pl.ds(base + kk * ZR, ZR)])
                if cnt_pass:
                    pltpu.sync_copy(zb16, accc.at[pl.ds(base + kk * ZR, ZR)])
            plsc.subcore_barrier()

            pltpu.async_copy(x_hbm.at[srcv.at[0]], bufs[0], gsem[0])

            def body(i, carry):
                j0 = 2 * i
                pltpu.make_async_copy(
                    x_hbm.at[pl.ds(0, CHUNK)], bufs[0], gsem[0]).wait()
                pltpu.async_copy(x_hbm.at[srcv.at[j0 + 1]], bufs[1], gsem[1])
                pltpu.sync_copy(bufs[0], acc.at[dstv.at[j0]], add=True)
                if cnt_pass:
                    pltpu.sync_copy(onesv, accc.at[dstv.at[j0]], add=True)

                @pl.when(j0 + 2 < NCH)
                def _():
                    pltpu.async_copy(
                        x_hbm.at[srcv.at[j0 + 2]], bufs[0], gsem[0])
                pltpu.make_async_copy(
                    x_hbm.at[pl.ds(0, CHUNK)], bufs[1], gsem[1]).wait()
                pltpu.sync_copy(bufs[1], acc.at[dstv.at[j0 + 1]], add=True)
                if cnt_pass:
                    pltpu.sync_copy(
                        onesv, accc.at[dstv.at[j0 + 1]], add=True)
                return carry

            lax.fori_loop(0, NCH // 2, body, 0)
            plsc.subcore_barrier()

            for kk in range(ROWS_PT // ZR):
                pltpu.sync_copy(acc.at[pl.ds(base + kk * ZR, ZR)], stg)
                pltpu.sync_copy(
                    stg, s_hbm[h].at[pl.ds(g * N_PAD + base + kk * ZR, ZR)])
                if cnt_pass:
                    pltpu.sync_copy(accc.at[pl.ds(base + kk * ZR, ZR)], stg16)
                    pltpu.sync_copy(
                        stg16,
                        cnt_hbm.at[pl.ds(g * N_PAD + base + kk * ZR, ZR)])

    return k(x_half, src2_pad, dst_pad, zeros_d, zeros_16, ones_16)


def _tc_layer(s0, s1, cnt, x, wlt, b, wrt):
    B = 1024
    NB = N_PAD // B

    def body(s0_ref, s1_ref, c_ref, x_ref, wl_ref, b_ref, wr_ref, o_ref):
        rcp = 1.0 / jnp.maximum(c_ref[:, 0:1], 1.0)
        agg = jnp.dot(s0_ref[...], wl_ref[0, :64],
                      preferred_element_type=jnp.float32)
        agg += jnp.dot(s1_ref[...], wl_ref[0, 64:],
                       preferred_element_type=jnp.float32)
        res = jnp.dot(x_ref[...], wr_ref[0], preferred_element_type=jnp.float32)
        o_ref[...] = jnp.tanh(agg * rcp + b_ref[0] + res)

    return pl.pallas_call(
        body,
        grid=(2, NB),
        in_specs=[
            pl.BlockSpec((B, 64), lambda g, i: (g * NB + i, 0)),
            pl.BlockSpec((B, 64), lambda g, i: (g * NB + i, 0)),
            pl.BlockSpec((B, 16), lambda g, i: (g * NB + i, 0)),
            pl.BlockSpec((B, D), lambda g, i: (g * NB + i, 0)),
            pl.BlockSpec((1, D, D), lambda g, i: (g, 0, 0)),
            pl.BlockSpec((1, 1, D), lambda g, i: (g, 0, 0)),
            pl.BlockSpec((1, D, D), lambda g, i: (g, 0, 0)),
        ],
        out_specs=pl.BlockSpec((B, D), lambda g, i: (g * NB + i, 0)),
        out_shape=jax.ShapeDtypeStruct((2 * N_PAD, D), jnp.float32),
    )(s0, s1, cnt, x, wlt, b, wrt)


def _pad_edges(ei, g):
    src2 = 2 * (ei[0] + g * N_PAD)
    dst = ei[1]
    src2 = src2.reshape(NT, EPT)
    dst = dst.reshape(NT, EPT)
    pad = EPT_PAD - EPT
    src2 = jnp.pad(src2, ((0, 0), (0, pad)), constant_values=2 * g * N_PAD)
    dst = jnp.pad(dst, ((0, 0), (0, pad)), constant_values=N)
    src2 = src2.reshape(NT, NCH, CHUNK)
    return jnp.stack([src2, src2 + 1]), dst.reshape(NT, NCH, CHUNK)


def kernel(x0, x1, edge_index0, edge_index1,
           g0_W1l, g0_b1, g0_W1r, g0_W2l, g0_b2, g0_W2r,
           g1_W1l, g1_b1, g1_W1r, g1_W2l, g1_b2, g1_W2r):
    pad = ((0, N_PAD - N), (0, 0))
    xs = jnp.concatenate([jnp.pad(x0, pad), jnp.pad(x1, pad)], axis=0)
    sg0, dg0 = _pad_edges(edge_index0, 0)
    sg1, dg1 = _pad_edges(edge_index1, 1)
    src2_pad = jnp.stack([sg0, sg1], axis=1)
    dst_pad = jnp.stack([dg0, dg1])

    zeros_d = jnp.zeros((ZR, 64), jnp.float32)
    zeros_16 = jnp.zeros((ZR, 16), jnp.float32)
    ones_16 = jnp.ones((CHUNK, 16), jnp.float32)

    w1lt = jnp.stack([g0_W1l.T, g1_W1l.T])
    w1rt = jnp.stack([g0_W1r.T, g1_W1r.T])
    b1 = jnp.stack([g0_b1, g1_b1])[:, None, :]
    w2lt = jnp.stack([g0_W2l.T, g1_W2l.T])
    w2rt = jnp.stack([g0_W2r.T, g1_W2r.T])
    b2 = jnp.stack([g0_b2, g1_b2])[:, None, :]

    xs_half = xs.reshape(4 * N_PAD, 64)
    sa, sb, cnt = _sc_seg_sum(xs_half, src2_pad, dst_pad, zeros_d, zeros_16,
                              ones_16, with_cnt=True)
    h1 = _tc_layer(sa, sb, cnt, xs, w1lt, b1, w1rt)
    sa2, sb2 = _sc_seg_sum(h1.reshape(4 * N_PAD, 64), src2_pad, dst_pad,
                           zeros_d, zeros_16, ones_16, with_cnt=False)
    out = _tc_layer(sa2, sb2, cnt, h1, w2lt, b2, w2rt)
    return out.reshape(2, N_PAD, D)[:, :N].reshape(2 * N, D)


# speedup vs baseline: 2.2140x; 1.0089x over previous
import functools

import jax
import jax.numpy as jnp
from jax import lax
from jax.experimental import pallas as pl
from jax.experimental.pallas import tpu as pltpu
from jax.experimental.pallas import tpu_sc as plsc

N = 10000
E = 320000
D = 128

NT = 16
CHUNK = 128
EPT = E // NT
NCH = 158
EPT_PAD = NCH * CHUNK
N_PAD = 10240
ROWS_PT = N_PAD // NT
ZR = 128


def _sc_seg_sum(x_half, src2_pad, dst_pad, zeros_d, zeros_16, ones_16,
                with_cnt):
    mesh = plsc.VectorSubcoreMesh(core_axis_name="c", subcore_axis_name="s")

    out_type = [jax.ShapeDtypeStruct((2 * N_PAD, 64), jnp.float32),
                jax.ShapeDtypeStruct((2 * N_PAD, 64), jnp.float32)]
    if with_cnt:
        out_type.append(jax.ShapeDtypeStruct((2 * N_PAD, 16), jnp.float32))

    scratch = [
        pltpu.VMEM((NCH, CHUNK), jnp.int32),
        pltpu.VMEM((NCH, CHUNK), jnp.int32),
        pltpu.VMEM((CHUNK, 64), jnp.float32),
        pltpu.VMEM((CHUNK, 64), jnp.float32),
        pltpu.VMEM((CHUNK, 16), jnp.float32),
        pltpu.VMEM((ZR, 64), jnp.float32),
        pltpu.VMEM((ZR, 16), jnp.float32),
        pltpu.VMEM((ZR, 64), jnp.float32),
        pltpu.VMEM((ZR, 16), jnp.float32),
        pltpu.VMEM_SHARED((N_PAD, 64), jnp.float32),
        pltpu.VMEM_SHARED((N_PAD, 16), jnp.float32),
    ] + [pltpu.SemaphoreType.DMA] * 2

    @functools.partial(
        pl.kernel, out_type=tuple(out_type), mesh=mesh,
        scratch_types=scratch, name="sc_seg_sum",
        compiler_params=pltpu.CompilerParams(use_tc_tiling_on_sc=False),
    )
    def k(x_hbm, src_hbm, dst_hbm, zd_hbm, z16_hbm, o16_hbm, *rest):
        if with_cnt:
            s_hbm = (rest[0], rest[1])
            cnt_hbm = rest[2]
            rest = rest[3:]
        else:
            s_hbm = (rest[0], rest[1])
            rest = rest[2:]
        (srcv, dstv, r0, r1, onesv, zb, zb16, stg, stg16,
         acc, accc, *sems) = rest
        bufs = [r0, r1]
        gsem = sems

        g = lax.axis_index("c")
        t = lax.axis_index("s")
        base = t * ROWS_PT

        pltpu.sync_copy(dst_hbm.at[g, t], dstv)
        pltpu.sync_copy(zd_hbm, zb)
        if with_cnt:
            pltpu.sync_copy(z16_hbm, zb16)
            pltpu.sync_copy(o16_hbm, onesv)

        for h in range(2):
            cnt_pass = with_cnt and h == 0
            pltpu.sync_copy(src_hbm.at[h, g, t], srcv)

            for kk in range(ROWS_PT // ZR):
                pltpu.sync_copy(zb, acc.at[pl.ds(base + kk * ZR, ZR)])
                if cnt_pass:
                    pltpu.sync_copy(zb16, accc.at[pl.ds(base + kk * ZR, ZR)])
            plsc.subcore_barrier()

            pltpu.async_copy(x_hbm.at[srcv.at[0]], bufs[0], gsem[0])

            def body(i, carry):
                j0 = 2 * i
                pltpu.async_copy(x_hbm.at[srcv.at[j0 + 1]], bufs[1], gsem[1])
                pltpu.make_async_copy(
                    x_hbm.at[pl.ds(0, CHUNK)], bufs[0], gsem[0]).wait()
                pltpu.sync_copy(bufs[0], acc.at[dstv.at[j0]], add=True)
                if cnt_pass:
                    pltpu.sync_copy(onesv, accc.at[dstv.at[j0]], add=True)

                @pl.when(j0 + 2 < NCH)
                def _():
                    pltpu.async_copy(
                        x_hbm.at[srcv.at[j0 + 2]], bufs[0], gsem[0])
                pltpu.make_async_copy(
                    x_hbm.at[pl.ds(0, CHUNK)], bufs[1], gsem[1]).wait()
                pltpu.sync_copy(bufs[1], acc.at[dstv.at[j0 + 1]], add=True)
                if cnt_pass:
                    pltpu.sync_copy(
                        onesv, accc.at[dstv.at[j0 + 1]], add=True)
                return carry

            lax.fori_loop(0, NCH // 2, body, 0)
            plsc.subcore_barrier()

            for kk in range(ROWS_PT // ZR):
                pltpu.sync_copy(acc.at[pl.ds(base + kk * ZR, ZR)], stg)
                pltpu.sync_copy(
                    stg, s_hbm[h].at[pl.ds(g * N_PAD + base + kk * ZR, ZR)])
                if cnt_pass:
                    pltpu.sync_copy(accc.at[pl.ds(base + kk * ZR, ZR)], stg16)
                    pltpu.sync_copy(
                        stg16,
                        cnt_hbm.at[pl.ds(g * N_PAD + base + kk * ZR, ZR)])

    return k(x_half, src2_pad, dst_pad, zeros_d, zeros_16, ones_16)


def _tc_layer(s0, s1, cnt, x, wlt, b, wrt):
    B = 1024
    NB = N_PAD // B

    def body(s0_ref, s1_ref, c_ref, x_ref, wl_ref, b_ref, wr_ref, o_ref):
        rcp = 1.0 / jnp.maximum(c_ref[:, 0:1], 1.0)
        agg = jnp.dot(s0_ref[...], wl_ref[0, :64],
                      preferred_element_type=jnp.float32)
        agg += jnp.dot(s1_ref[...], wl_ref[0, 64:],
                       preferred_element_type=jnp.float32)
        res = jnp.dot(x_ref[...], wr_ref[0], preferred_element_type=jnp.float32)
        o_ref[...] = jnp.tanh(agg * rcp + b_ref[0] + res)

    return pl.pallas_call(
        body,
        grid=(2, NB),
        in_specs=[
            pl.BlockSpec((B, 64), lambda g, i: (g * NB + i, 0)),
            pl.BlockSpec((B, 64), lambda g, i: (g * NB + i, 0)),
            pl.BlockSpec((B, 16), lambda g, i: (g * NB + i, 0)),
            pl.BlockSpec((B, D), lambda g, i: (g * NB + i, 0)),
            pl.BlockSpec((1, D, D), lambda g, i: (g, 0, 0)),
            pl.BlockSpec((1, 1, D), lambda g, i: (g, 0, 0)),
            pl.BlockSpec((1, D, D), lambda g, i: (g, 0, 0)),
        ],
        out_specs=pl.BlockSpec((B, D), lambda g, i: (g * NB + i, 0)),
        out_shape=jax.ShapeDtypeStruct((2 * N_PAD, D), jnp.float32),
    )(s0, s1, cnt, x, wlt, b, wrt)


def _pad_edges(ei, g):
    src2 = 2 * (ei[0] + g * N_PAD)
    dst = ei[1]
    src2 = src2.reshape(NT, EPT)
    dst = dst.reshape(NT, EPT)
    pad = EPT_PAD - EPT
    src2 = jnp.pad(src2, ((0, 0), (0, pad)), constant_values=2 * g * N_PAD)
    dst = jnp.pad(dst, ((0, 0), (0, pad)), constant_values=N)
    src2 = src2.reshape(NT, NCH, CHUNK)
    return jnp.stack([src2, src2 + 1]), dst.reshape(NT, NCH, CHUNK)


def kernel(x0, x1, edge_index0, edge_index1,
           g0_W1l, g0_b1, g0_W1r, g0_W2l, g0_b2, g0_W2r,
           g1_W1l, g1_b1, g1_W1r, g1_W2l, g1_b2, g1_W2r):
    pad = ((0, N_PAD - N), (0, 0))
    xs = jnp.concatenate([jnp.pad(x0, pad), jnp.pad(x1, pad)], axis=0)
    sg0, dg0 = _pad_edges(edge_index0, 0)
    sg1, dg1 = _pad_edges(edge_index1, 1)
    src2_pad = jnp.stack([sg0, sg1], axis=1)
    dst_pad = jnp.stack([dg0, dg1])

    zeros_d = jnp.zeros((ZR, 64), jnp.float32)
    zeros_16 = jnp.zeros((ZR, 16), jnp.float32)
    ones_16 = jnp.ones((CHUNK, 16), jnp.float32)

    w1lt = jnp.stack([g0_W1l.T, g1_W1l.T])
    w1rt = jnp.stack([g0_W1r.T, g1_W1r.T])
    b1 = jnp.stack([g0_b1, g1_b1])[:, None, :]
    w2lt = jnp.stack([g0_W2l.T, g1_W2l.T])
    w2rt = jnp.stack([g0_W2r.T, g1_W2r.T])
    b2 = jnp.stack([g0_b2, g1_b2])[:, None, :]

    xs_half = xs.reshape(4 * N_PAD, 64)
    sa, sb, cnt = _sc_seg_sum(xs_half, src2_pad, dst_pad, zeros_d, zeros_16,
                              ones_16, with_cnt=True)
    h1 = _tc_layer(sa, sb, cnt, xs, w1lt, b1, w1rt)
    sa2, sb2 = _sc_seg_sum(h1.reshape(4 * N_PAD, 64), src2_pad, dst_pad,
                           zeros_d, zeros_16, ones_16, with_cnt=False)
    out = _tc_layer(sa2, sb2, cnt, h1, w2lt, b2, w2rt)
    return out.reshape(2, N_PAD, D)[:, :N].reshape(2 * N, D)
